# Initial kernel scaffold; baseline (speedup 1.0000x reference)
import jax
import jax.numpy as jnp
from jax.experimental import pallas as pl


def kernel(h, edge_index_0, edge_index_1, W0, al0, ar0, b0, W1, al1, ar1, b1, P1, pb1, P2):
    raise NotImplementedError("write your pallas kernel here")


# speedup vs baseline: 40.9223x; 40.9223x over previous
import functools

import jax
import jax.numpy as jnp
from jax import lax
from jax.experimental import pallas as pl
from jax.experimental.pallas import tpu as pltpu
from jax.experimental.pallas import tpu_sc as plsc

_N = 10000
_E = 320000
_H = 8
_OUT = 16
_D = 128
_DE = _H * _OUT

_NTILES = 16
_BLK = 128
_ET = 20096
_NBLK = _ET // _BLK
_EPAD = _NTILES * _ET
_ROWS = 10240
_ZROWS = _ROWS // _NTILES
_OROWS = _ROWS // _NTILES

_BN = 1000
_BNC = 1024


def _tc_feat_body(h_ref, w_ref, a_ref, feat_ref, eler_ref):
    f = jnp.dot(h_ref[...], w_ref[0], preferred_element_type=jnp.float32)
    feat_ref[0] = f
    eler_ref[0] = jnp.dot(f, a_ref[0], preferred_element_type=jnp.float32)


def _tc_feat(h, Wc, Ac):
    grid = (2, _N // _BN)
    return pl.pallas_call(
        _tc_feat_body,
        grid=grid,
        in_specs=[
            pl.BlockSpec((_BN, _D), lambda c, i: (i, 0)),
            pl.BlockSpec((1, _D, _DE), lambda c, i: (c, 0, 0)),
            pl.BlockSpec((1, _DE, 2 * _H), lambda c, i: (c, 0, 0)),
        ],
        out_specs=[
            pl.BlockSpec((1, _BN, _DE), lambda c, i: (c, i, 0)),
            pl.BlockSpec((1, _BN, 2 * _H), lambda c, i: (c, i, 0)),
        ],
        out_shape=[
            jax.ShapeDtypeStruct((2, _N, _DE), jnp.float32),
            jax.ShapeDtypeStruct((2, _N, 2 * _H), jnp.float32),
        ],
    )(h, Wc, Ac)


def _sc_edge_body(featf, elerf, srcs, dstsg, dstsl, zacc, zs,
                  acc_out, s_out,
                  src_i, dstg_i, dstl_i, frows, eler_s, eler_d, w_s,
                  acc_sh, s_sh, sem):
    c = lax.axis_index("c")
    t = lax.axis_index("s")

    pltpu.sync_copy(zacc, acc_sh.at[pl.ds(t * _ZROWS, _ZROWS)])
    pltpu.sync_copy(zs, s_sh.at[pl.ds(t * _ZROWS, _ZROWS)])
    plsc.subcore_barrier()

    ebase = t * _ET

    def block_body(b, carry):
        off = ebase + b * _BLK
        pltpu.sync_copy(srcs.at[c, pl.ds(off, _BLK)], src_i)
        pltpu.sync_copy(dstsg.at[c, pl.ds(off, _BLK)], dstg_i)
        pltpu.sync_copy(dstsl.at[c, pl.ds(off, _BLK)], dstl_i)
        pltpu.async_copy(featf.at[src_i], frows, sem).wait()
        pltpu.async_copy(elerf.at[src_i], eler_s, sem).wait()
        pltpu.async_copy(elerf.at[dstg_i], eler_d, sem).wait()

        for g in range(_BLK // 16):
            ids = lax.iota(jnp.int32, 16) + g * 16
            for h in range(_H):
                hv = jnp.full((16,), h, jnp.int32)
                el = plsc.load_gather(eler_s, [ids, hv])
                er = plsc.load_gather(eler_d, [ids, hv + _H])
                e = el + er
                e = jnp.where(e > 0, e, 0.2 * e)
                plsc.store_scatter(w_s, [ids, hv], jnp.exp(e))

        def edge_body(i, carry2):
            iv = jnp.full((16,), i, jnp.int32)
            for h in range(_H):
                wsp = plsc.load_gather(w_s, [iv, jnp.full((16,), h, jnp.int32)])
                frows[i, pl.ds(h * 16, 16)] = frows[i, pl.ds(h * 16, 16)] * wsp
            return carry2

        lax.fori_loop(0, _BLK, edge_body, 0)

        pltpu.sync_copy(frows, acc_sh.at[dstl_i], add=True)
        pltpu.sync_copy(w_s, s_sh.at[dstl_i], add=True)
        return carry

    lax.fori_loop(0, _NBLK, block_body, 0)
    plsc.subcore_barrier()

    pltpu.sync_copy(acc_sh.at[pl.ds(t * _OROWS, _OROWS)],
                    acc_out.at[c, pl.ds(t * _OROWS, _OROWS)])
    pltpu.sync_copy(s_sh.at[pl.ds(t * _OROWS, _OROWS)],
                    s_out.at[c, pl.ds(t * _OROWS, _OROWS)])


def _build_sc_edge():
    return functools.partial(
        pl.kernel,
        out_type=(
            jax.ShapeDtypeStruct((2, _ROWS, _DE), jnp.float32),
            jax.ShapeDtypeStruct((2, _ROWS, _H), jnp.float32),
        ),
        mesh=plsc.VectorSubcoreMesh(core_axis_name="c", subcore_axis_name="s",
                                    num_cores=2, num_subcores=_NTILES),
        compiler_params=pltpu.CompilerParams(needs_layout_passes=False,
                                             use_tc_tiling_on_sc=False),
        scratch_types=[
        pltpu.VMEM((_BLK,), jnp.int32),
        pltpu.VMEM((_BLK,), jnp.int32),
        pltpu.VMEM((_BLK,), jnp.int32),
        pltpu.VMEM((_BLK, _DE), jnp.float32),
        pltpu.VMEM((_BLK, 2 * _H), jnp.float32),
        pltpu.VMEM((_BLK, 2 * _H), jnp.float32),
        pltpu.VMEM((_BLK, _H), jnp.float32),
            pltpu.VMEM_SHARED((_ROWS, _DE), jnp.float32),
            pltpu.VMEM_SHARED((_ROWS, _H), jnp.float32),
            pltpu.SemaphoreType.DMA,
        ],
    )(_sc_edge_body)


def _tc_norm_body(acc_ref, s_ref, b_ref, r_ref, p1_ref, pb1_ref, p2_ref,
                  z_ref, wsum_ref):
    c = pl.program_id(0)
    i = pl.program_id(1)
    s = s_ref[0]
    srec = jnp.where(s > 0, 1.0 / jnp.where(s > 0, s, 1.0), 0.0)
    sexp = jnp.dot(srec, r_ref[...], preferred_element_type=jnp.float32)
    rst = acc_ref[0] * sexp + b_ref[pl.ds(c, 1), :]
    z = jnp.where(rst > 0, rst, jnp.exp(jnp.minimum(rst, 0.0)) - 1.0)
    z_ref[0] = z
    q = jnp.tanh(jnp.dot(z, p1_ref[...], preferred_element_type=jnp.float32)
                 + pb1_ref[...])
    grow = i * _BNC + lax.broadcasted_iota(jnp.int32, (_BNC, 1), 0)
    part = jnp.sum(jnp.where(grow < _N, q * p2_ref[...], 0.0))

    @pl.when(jnp.logical_and(c == 0, i == 0))
    def _():
        wsum_ref[...] = jnp.zeros_like(wsum_ref)

    row = lax.broadcasted_iota(jnp.int32, (2, _DE), 0)
    wsum_ref[...] += jnp.where(row == c, part, 0.0)


def _tc_norm(accf, sf, bc, R, P1, pb1r, P2r):
    grid = (2, _ROWS // _BNC)
    return pl.pallas_call(
        _tc_norm_body,
        grid=grid,
        in_specs=[
            pl.BlockSpec((1, _BNC, _DE), lambda c, i: (c, i, 0)),
            pl.BlockSpec((1, _BNC, _H), lambda c, i: (c, i, 0)),
            pl.BlockSpec((2, _DE), lambda c, i: (0, 0)),
            pl.BlockSpec((_H, _DE), lambda c, i: (0, 0)),
            pl.BlockSpec((_DE, _DE), lambda c, i: (0, 0)),
            pl.BlockSpec((1, _DE), lambda c, i: (0, 0)),
            pl.BlockSpec((1, _DE), lambda c, i: (0, 0)),
        ],
        out_specs=[
            pl.BlockSpec((1, _BNC, _DE), lambda c, i: (c, i, 0)),
            pl.BlockSpec((2, _DE), lambda c, i: (0, 0)),
        ],
        out_shape=[
            jax.ShapeDtypeStruct((2, _ROWS, _DE), jnp.float32),
            jax.ShapeDtypeStruct((2, _DE), jnp.float32),
        ],
    )(accf, sf, bc, R, P1, pb1r, P2r)


def _tc_mix_body(w_ref, z_ref, out_ref):
    w = w_ref[:, 0:1] * (1.0 / _N)
    m = jnp.max(w)
    ex = jnp.exp(w - m)
    beta = ex / jnp.sum(ex)
    out_ref[...] = (z_ref[0] * beta[0:1, 0:1] + z_ref[1] * beta[1:2, 0:1])


def _tc_mix(wsum, z):
    grid = (_N // _BN,)
    return pl.pallas_call(
        _tc_mix_body,
        grid=grid,
        in_specs=[
            pl.BlockSpec((2, _DE), lambda i: (0, 0)),
            pl.BlockSpec((2, _BN, _DE), lambda i: (0, i, 0)),
        ],
        out_specs=pl.BlockSpec((_BN, _DE), lambda i: (i, 0)),
        out_shape=jax.ShapeDtypeStruct((_N, _DE), jnp.float32),
    )(wsum, z)


def _fold_attn(al, ar):
    eye = jnp.eye(_H, dtype=jnp.float32)
    Al = (al[:, :, None] * eye[:, None, :]).reshape(_DE, _H)
    Ar = (ar[:, :, None] * eye[:, None, :]).reshape(_DE, _H)
    return jnp.concatenate([Al, Ar], axis=1)


def _pad_edges(ei, c):
    pad = _EPAD - _E
    src_g = jnp.concatenate(
        [ei[0] + c * _N, jnp.full((pad,), c * _N, jnp.int32)])
    dst_g = jnp.concatenate(
        [ei[1] + c * _N, jnp.full((pad,), c * _N, jnp.int32)])
    dst_l = jnp.concatenate(
        [ei[1], jnp.full((pad,), _N, jnp.int32)])
    return src_g, dst_g, dst_l


def kernel(h, edge_index_0, edge_index_1, W0, al0, ar0, b0,
           W1, al1, ar1, b1, P1, pb1, P2):
    Wc = jnp.stack([W0, W1])
    Ac = jnp.stack([_fold_attn(al0, ar0), _fold_attn(al1, ar1)])
    featc, elerc = _tc_feat(h, Wc, Ac)
    featf = featc.reshape(2 * _N, _DE)
    elerf = elerc.reshape(2 * _N, 2 * _H)

    s0 = _pad_edges(edge_index_0, 0)
    s1 = _pad_edges(edge_index_1, 1)
    srcs = jnp.stack([s0[0], s1[0]])
    dstsg = jnp.stack([s0[1], s1[1]])
    dstsl = jnp.stack([s0[2], s1[2]])

    zacc = jnp.zeros((_ZROWS, _DE), jnp.float32)
    zs = jnp.zeros((_ZROWS, _H), jnp.float32)

    accf, sf = _build_sc_edge()(featf, elerf, srcs, dstsg, dstsl, zacc, zs)

    bc = jnp.stack([b0, b1])
    R = (jnp.eye(_H, dtype=jnp.float32)[:, :, None]
         * jnp.ones((1, 1, _OUT), jnp.float32)).reshape(_H, _DE)
    z, wsum = _tc_norm(accf, sf, bc, R, P1, pb1.reshape(1, _DE),
                       P2.reshape(1, _DE))
    return _tc_mix(wsum, z)


# speedup vs baseline: 46.5845x; 1.1384x over previous
import functools

import jax
import jax.numpy as jnp
from jax import lax
from jax.experimental import pallas as pl
from jax.experimental.pallas import tpu as pltpu
from jax.experimental.pallas import tpu_sc as plsc

_N = 10000
_E = 320000
_H = 8
_OUT = 16
_D = 128
_DE = _H * _OUT
_FX = 144
_ER = 8

_NTILES = 16
_BLK = 112
_NBLK = 180
_NSLOT = _NBLK + 2
_ET = _NBLK * _BLK
_EPAD = _NTILES * _ET
_NP = 10112
_ZROWS = _NP // _NTILES

_BN = 1000
_BNC = 632


def _tc_feat_body(h_ref, w_ref, a_ref, featx_ref, erx_ref):
    f = jnp.dot(h_ref[...], w_ref[0], preferred_element_type=jnp.float32)
    eb = jnp.dot(f, a_ref[0], preferred_element_type=jnp.float32)
    zpad = jnp.zeros((_BN, 8), jnp.float32)
    featx_ref[0] = jnp.concatenate([f, eb[:, :_H], zpad], axis=1)
    erx_ref[0] = eb[:, _H:]


def _tc_feat(h, Wc, Ac):
    grid = (2, _N // _BN)
    return pl.pallas_call(
        _tc_feat_body,
        grid=grid,
        in_specs=[
            pl.BlockSpec((_BN, _D), lambda c, i: (i, 0)),
            pl.BlockSpec((1, _D, _DE), lambda c, i: (c, 0, 0)),
            pl.BlockSpec((1, _DE, 2 * _H), lambda c, i: (c, 0, 0)),
        ],
        out_specs=[
            pl.BlockSpec((1, _BN, _FX), lambda c, i: (c, i, 0)),
            pl.BlockSpec((1, _BN, _ER), lambda c, i: (c, i, 0)),
        ],
        out_shape=[
            jax.ShapeDtypeStruct((2, _NP, _FX), jnp.float32),
            jax.ShapeDtypeStruct((2, _NP, _ER), jnp.float32),
        ],
    )(h, Wc, Ac)


def _sc_edge_body(featx, erx, eidx, zacc,
                  acc_out,
                  idx0, idx1, er0, er1, msg0, msg1,
                  acc_sh, sg0, sg1):
    c = lax.axis_index("c")
    t = lax.axis_index("s")

    idxb = (idx0, idx1)
    erb = (er0, er1)
    msgb = (msg0, msg1)
    sg = (sg0, sg1)

    pltpu.sync_copy(zacc, acc_sh.at[pl.ds(t * _ZROWS, _ZROWS)])
    plsc.subcore_barrier()

    sbase = t * _NSLOT

    def issue_idx_sync(b, s):
        pltpu.sync_copy(eidx.at[c, sbase + b], idxb[s])

    def issue_gathers(s):
        pltpu.async_copy(featx.at[idxb[s].at[0]], msgb[s], sg[s])
        pltpu.async_copy(erx.at[idxb[s].at[1]], erb[s], sg[s])

    def wait_gathers(s):
        pltpu.make_async_copy(featx.at[pl.ds(0, _BLK)], msgb[s], sg[s]).wait()
        pltpu.make_async_copy(erx.at[pl.ds(0, _BLK)], erb[s], sg[s]).wait()

    for s in (0, 1):
        issue_idx_sync(s, s)
        issue_gathers(s)

    iota16 = lax.iota(jnp.int32, 16)

    def compute(s):
        er = erb[s]
        msg = msgb[s]
        for g in range(_BLK // 16):
            ids = iota16 + g * 16
            for h in range(_H):
                el_h = plsc.load_gather(msg, [ids, jnp.full((16,), _DE + h,
                                                            jnp.int32)])
                er_h = plsc.load_gather(er, [ids, jnp.full((16,), h,
                                                           jnp.int32)])
                e = el_h + er_h
                e = jnp.where(e > 0, e, 0.2 * e)
                plsc.store_scatter(msg, [ids, jnp.full((16,), _DE + h,
                                                       jnp.int32)],
                                   jnp.exp(e))

        def edge_body(i, carry):
            iv = jnp.full((16,), i, jnp.int32)
            for h in range(_H):
                wsp = plsc.load_gather(msg, [iv, jnp.full((16,), _DE + h,
                                                          jnp.int32)])
                msg[i, pl.ds(h * 16, 16)] = msg[i, pl.ds(h * 16, 16)] * wsp
            return carry

        lax.fori_loop(0, _BLK, edge_body, 0)

    def visit(b, s):
        wait_gathers(s)
        compute(s)
        pltpu.sync_copy(msgb[s], acc_sh.at[idxb[s].at[2]], add=True)
        issue_idx_sync(b + 2, s)
        issue_gathers(s)

    def pair_body(j, carry):
        visit(2 * j, 0)
        visit(2 * j + 1, 1)
        return carry

    lax.fori_loop(0, _NBLK // 2, pair_body, 0)

    wait_gathers(0)
    wait_gathers(1)
    plsc.subcore_barrier()

    pltpu.sync_copy(acc_sh.at[pl.ds(t * _ZROWS, _ZROWS)],
                    acc_out.at[c, pl.ds(t * _ZROWS, _ZROWS)])


def _build_sc_edge():
    return functools.partial(
        pl.kernel,
        out_type=jax.ShapeDtypeStruct((2, _NP, _FX), jnp.float32),
        mesh=plsc.VectorSubcoreMesh(core_axis_name="c", subcore_axis_name="s",
                                    num_cores=2, num_subcores=_NTILES),
        compiler_params=pltpu.CompilerParams(needs_layout_passes=False,
                                             use_tc_tiling_on_sc=False),
        scratch_types=[
            pltpu.VMEM((3, _BLK), jnp.int32),
            pltpu.VMEM((3, _BLK), jnp.int32),
            pltpu.VMEM((_BLK, _ER), jnp.float32),
            pltpu.VMEM((_BLK, _ER), jnp.float32),
            pltpu.VMEM((_BLK, _FX), jnp.float32),
            pltpu.VMEM((_BLK, _FX), jnp.float32),
            pltpu.VMEM_SHARED((_NP, _FX), jnp.float32),
            pltpu.SemaphoreType.DMA,
            pltpu.SemaphoreType.DMA,
        ],
    )(_sc_edge_body)


def _tc_norm_body(acc_ref, b_ref, r_ref, p1_ref, pb1_ref, p2_ref,
                  z_ref, wsum_ref):
    c = pl.program_id(0)
    i = pl.program_id(1)
    blk = acc_ref[0]
    acc = blk[:, :_DE]
    s = blk[:, _DE:_DE + _H]
    srec = jnp.where(s > 0, 1.0 / jnp.where(s > 0, s, 1.0), 0.0)
    sexp = jnp.dot(srec, r_ref[...], preferred_element_type=jnp.float32)
    rst = acc * sexp + b_ref[pl.ds(c, 1), :]
    z = jnp.where(rst > 0, rst, jnp.exp(jnp.minimum(rst, 0.0)) - 1.0)
    z_ref[0] = z
    q = jnp.tanh(jnp.dot(z, p1_ref[...], preferred_element_type=jnp.float32)
                 + pb1_ref[...])
    grow = i * _BNC + lax.broadcasted_iota(jnp.int32, (_BNC, 1), 0)
    part = jnp.sum(jnp.where(grow < _N, q * p2_ref[...], 0.0))

    @pl.when(jnp.logical_and(c == 0, i == 0))
    def _():
        wsum_ref[...] = jnp.zeros_like(wsum_ref)

    row = lax.broadcasted_iota(jnp.int32, (2, _DE), 0)
    wsum_ref[...] += jnp.where(row == c, part, 0.0)


def _tc_norm(accf, bc, R, P1, pb1r, P2r):
    grid = (2, _NP // _BNC)
    return pl.pallas_call(
        _tc_norm_body,
        grid=grid,
        in_specs=[
            pl.BlockSpec((1, _BNC, _FX), lambda c, i: (c, i, 0)),
            pl.BlockSpec((2, _DE), lambda c, i: (0, 0)),
            pl.BlockSpec((_H, _DE), lambda c, i: (0, 0)),
            pl.BlockSpec((_DE, _DE), lambda c, i: (0, 0)),
            pl.BlockSpec((1, _DE), lambda c, i: (0, 0)),
            pl.BlockSpec((1, _DE), lambda c, i: (0, 0)),
        ],
        out_specs=[
            pl.BlockSpec((1, _BNC, _DE), lambda c, i: (c, i, 0)),
            pl.BlockSpec((2, _DE), lambda c, i: (0, 0)),
        ],
        out_shape=[
            jax.ShapeDtypeStruct((2, _NP, _DE), jnp.float32),
            jax.ShapeDtypeStruct((2, _DE), jnp.float32),
        ],
    )(accf, bc, R, P1, pb1r, P2r)


def _tc_mix_body(w_ref, z_ref, out_ref):
    w = w_ref[:, 0:1] * (1.0 / _N)
    m = jnp.max(w)
    ex = jnp.exp(w - m)
    beta = ex / jnp.sum(ex)
    out_ref[...] = (z_ref[0] * beta[0:1, 0:1] + z_ref[1] * beta[1:2, 0:1])


def _tc_mix(wsum, z):
    grid = (_N // _BN,)
    return pl.pallas_call(
        _tc_mix_body,
        grid=grid,
        in_specs=[
            pl.BlockSpec((2, _DE), lambda i: (0, 0)),
            pl.BlockSpec((2, _BN, _DE), lambda i: (0, i, 0)),
        ],
        out_specs=pl.BlockSpec((_BN, _DE), lambda i: (i, 0)),
        out_shape=jax.ShapeDtypeStruct((_N, _DE), jnp.float32),
    )(wsum, z)


def _fold_attn(al, ar):
    eye = jnp.eye(_H, dtype=jnp.float32)
    Al = (al[:, :, None] * eye[:, None, :]).reshape(_DE, _H)
    Ar = (ar[:, :, None] * eye[:, None, :]).reshape(_DE, _H)
    return jnp.concatenate([Al, Ar], axis=1)


def _build_eidx(ei, c):
    pad = _EPAD - _E
    src_g = jnp.concatenate(
        [ei[0], jnp.zeros((pad,), jnp.int32)]) + c * _NP
    dst_l = jnp.concatenate(
        [ei[1], jnp.full((pad,), _N, jnp.int32)])
    dst_g = dst_l + c * _NP
    arr = jnp.stack([src_g, dst_g, dst_l])
    arr = arr.reshape(3, _NTILES, _NBLK, _BLK).transpose(1, 2, 0, 3)
    dummy = jnp.zeros((_NTILES, 2, 3, _BLK), jnp.int32)
    arr = jnp.concatenate([arr, dummy], axis=1)
    return arr.reshape(_NTILES * _NSLOT, 3, _BLK)


def kernel(h, edge_index_0, edge_index_1, W0, al0, ar0, b0,
           W1, al1, ar1, b1, P1, pb1, P2):
    Wc = jnp.stack([W0, W1])
    Ac = jnp.stack([_fold_attn(al0, ar0), _fold_attn(al1, ar1)])
    featc, erc = _tc_feat(h, Wc, Ac)
    featx = featc.reshape(2 * _NP, _FX)
    erx = erc.reshape(2 * _NP, _ER)

    eidx = jnp.stack([_build_eidx(edge_index_0, 0),
                      _build_eidx(edge_index_1, 1)])

    zacc = jnp.zeros((_ZROWS, _FX), jnp.float32)

    accf = _build_sc_edge()(featx, erx, eidx, zacc)

    bc = jnp.stack([b0, b1])
    R = (jnp.eye(_H, dtype=jnp.float32)[:, :, None]
         * jnp.ones((1, 1, _OUT), jnp.float32)).reshape(_H, _DE)
    z, wsum = _tc_norm(accf, bc, R, P1, pb1.reshape(1, _DE),
                       P2.reshape(1, _DE))
    return _tc_mix(wsum, z)
